# Initial kernel scaffold; baseline (speedup 1.0000x reference)
import jax
import jax.numpy as jnp
from jax.experimental import pallas as pl


def kernel(x, edge_index, edge_attr, W1, b1, W2, b2, gamma, beta):
    raise NotImplementedError("write your pallas kernel here")


# speedup vs baseline: 4.3938x; 4.3938x over previous
import functools

import jax
import jax.numpy as jnp
from jax import lax
from jax.experimental import pallas as pl
from jax.experimental.pallas import tpu as pltpu
from jax.experimental.pallas import tpu_sc as plsc

N_NODES = 10000
D = 128
BN_EPS = 1e-5

NC = 2
NS = 16
NW = NC * NS
CHUNK = 128
N_PAD = 10240
ROWS_PER_TILE = N_PAD // NS


def _segment_sum_sc(src_pad, dst_pad, x):
    e_pad = src_pad.shape[0]
    e_per_w = e_pad // NW
    n_chunks = e_per_w // CHUNK
    mesh = plsc.VectorSubcoreMesh(core_axis_name="c", subcore_axis_name="s")

    @functools.partial(
        pl.kernel,
        mesh=mesh,
        out_type=jax.ShapeDtypeStruct((NC, N_PAD, D), jnp.float32),
        scratch_types=[
            pltpu.VMEM((CHUNK,), jnp.int32),
            pltpu.VMEM((CHUNK,), jnp.int32),
            pltpu.VMEM((CHUNK, D), jnp.float32),
            pltpu.VMEM_SHARED((N_PAD, D), jnp.float32),
            pltpu.SemaphoreType.DMA,
        ],
    )
    def seg_sum(src_hbm, dst_hbm, x_hbm, out_hbm, sidx, didx, rows, acc, sem):
        c = lax.axis_index("c")
        s = lax.axis_index("s")
        wid = s * NC + c

        zero = jnp.zeros((16,), jnp.float32)

        def zrow(i, _):
            def zcol(j, _):
                rows[i, pl.ds(j * 16, 16)] = zero
                return 0
            return lax.fori_loop(0, D // 16, zcol, 0)

        lax.fori_loop(0, CHUNK, zrow, 0)

        def zacc(i, _):
            pltpu.sync_copy(rows, acc.at[pl.ds(s * ROWS_PER_TILE + i * CHUNK, CHUNK)])
            return 0

        lax.fori_loop(0, ROWS_PER_TILE // CHUNK, zacc, 0)
        plsc.subcore_barrier()

        base = wid * e_per_w

        def body(j, _):
            off = base + j * CHUNK
            pltpu.sync_copy(src_hbm.at[pl.ds(off, CHUNK)], sidx)
            pltpu.sync_copy(dst_hbm.at[pl.ds(off, CHUNK)], didx)
            pltpu.async_copy(x_hbm.at[sidx], rows, sem).wait()
            pltpu.sync_copy(rows, acc.at[didx], add=True)
            return 0

        lax.fori_loop(0, n_chunks, body, 0)
        plsc.subcore_barrier()

        pltpu.sync_copy(
            acc.at[pl.ds(s * ROWS_PER_TILE, ROWS_PER_TILE)],
            out_hbm.at[c, pl.ds(s * ROWS_PER_TILE, ROWS_PER_TILE)],
        )

    return seg_sum(src_pad, dst_pad, x)


def _mlp_bn_tc(x, agg2, W1, b1, W2, b2, gamma, beta):
    def body(x_ref, agg_ref, w1_ref, b1_ref, w2_ref, b2_ref, g_ref, be_ref, out_ref):
        h = x_ref[...] + agg_ref[0, :N_NODES, :] + agg_ref[1, :N_NODES, :]
        h = jnp.dot(h, w1_ref[...], preferred_element_type=jnp.float32) + b1_ref[...]
        h = jnp.maximum(h, 0.0)
        h = jnp.dot(h, w2_ref[...], preferred_element_type=jnp.float32) + b2_ref[...]
        h = jnp.maximum(h, 0.0)
        mean = jnp.mean(h, axis=0, keepdims=True)
        var = jnp.mean((h - mean) ** 2, axis=0, keepdims=True)
        inv = lax.rsqrt(var + BN_EPS)
        out_ref[...] = g_ref[...] * (h - mean) * inv + be_ref[...]

    return pl.pallas_call(
        body,
        out_shape=jax.ShapeDtypeStruct((N_NODES, D), jnp.float32),
    )(x, agg2, W1, b1, W2, b2, gamma, beta)


def kernel(x, edge_index, edge_attr, W1, b1, W2, b2, gamma, beta):
    del edge_attr
    src = edge_index[0].astype(jnp.int32)
    dst = edge_index[1].astype(jnp.int32)
    e = src.shape[0]
    step = NW * CHUNK
    e_pad = ((e + step - 1) // step) * step
    pad = e_pad - e
    if pad:
        src = jnp.concatenate([src, jnp.zeros((pad,), jnp.int32)])
        dst = jnp.concatenate([dst, jnp.full((pad,), N_NODES, jnp.int32)])

    agg2 = _segment_sum_sc(src, dst, x)
    return _mlp_bn_tc(
        x, agg2, W1, b1.reshape(1, D), W2, b2.reshape(1, D),
        gamma.reshape(1, D), beta.reshape(1, D),
    )
